# Initial kernel scaffold; baseline (speedup 1.0000x reference)
import jax
import jax.numpy as jnp
from jax.experimental import pallas as pl


def kernel(inputs, embedding, reset):
    raise NotImplementedError("write your pallas kernel here")


# speedup vs baseline: 1.3115x; 1.3115x over previous
import jax
import jax.numpy as jnp
from jax.experimental import pallas as pl
from jax.experimental.pallas import tpu as pltpu
from jax.experimental.pallas import tpu_sc as plsc

K = 8192
D = 256
N = 8 * 32 * 32
NB = 256
NBLK = N // NB
GW = 128
COMMIT = 0.25


def _vq_main(x_ref, x2_ref, et_ref, e2_ref, idx_ref, enc_ref, counts_ref):
    i = pl.program_id(0)
    mm = jnp.dot(x_ref[...], et_ref[...], preferred_element_type=jnp.float32)
    d = (x2_ref[...] + e2_ref[...]) - 2.0 * mm
    vmin = jnp.min(d, axis=1, keepdims=True)
    iota = jax.lax.broadcasted_iota(jnp.int32, d.shape, 1)
    idx = jnp.min(jnp.where(d == vmin, iota, K), axis=1)
    idx_ref[...] = idx.reshape(1, 1, NB)
    enc = (iota == idx[:, None]).astype(jnp.float32)
    enc_ref[...] = enc

    @pl.when(i == 0)
    def _():
        counts_ref[...] = jnp.zeros_like(counts_ref)

    counts_ref[...] += jnp.sum(enc, axis=0, keepdims=True)


def _sc_gather(emb_hbm, i_hbm, o_hbm):
    def body(i_vmem, o_vmem):
        pltpu.sync_copy(emb_hbm.at[i_vmem.at[0]], o_vmem)

    pltpu.emit_pipeline(
        body,
        grid=(N // GW,),
        in_specs=[pl.BlockSpec((1, GW), index_map=lambda i: (0, i))],
        out_specs=[pl.BlockSpec((GW, D), index_map=lambda i: (i, 0))],
        core_axis_name=("core", "subcore"),
        dimension_semantics=(pltpu.PARALLEL,),
    )(i_hbm, o_hbm)


def _finalize(x_ref, q_ref, counts_ref, qst_ref, loss_ref, perp_ref):
    x = x_ref[...]
    q = q_ref[...]
    dq = q - x
    qst_ref[...] = x + dq
    mse = jnp.mean(dq * dq)
    loss_ref[...] = (mse + COMMIT * mse).reshape(1, 1)
    p = counts_ref[...] * (1.0 / N)
    ent = jnp.sum(p * jnp.log(p + 1e-10))
    perp_ref[...] = jnp.exp(-ent).reshape(1, 1)


def kernel(inputs, embedding, reset):
    del reset
    x = jnp.transpose(inputs, (0, 2, 3, 1))
    input_shape = x.shape
    flat = x.reshape(-1, D)
    x2 = jnp.sum(flat ** 2, axis=1, keepdims=True)
    e2 = jnp.sum(embedding ** 2, axis=1).reshape(1, K)
    et = embedding.T

    idx3, enc, counts = pl.pallas_call(
        _vq_main,
        grid=(NBLK,),
        in_specs=[
            pl.BlockSpec((NB, D), lambda i: (i, 0)),
            pl.BlockSpec((NB, 1), lambda i: (i, 0)),
            pl.BlockSpec((D, K), lambda i: (0, 0)),
            pl.BlockSpec((1, K), lambda i: (0, 0)),
        ],
        out_specs=[
            pl.BlockSpec((1, 1, NB), lambda i: (i, 0, 0)),
            pl.BlockSpec((NB, K), lambda i: (i, 0)),
            pl.BlockSpec((1, K), lambda i: (0, 0)),
        ],
        out_shape=[
            jax.ShapeDtypeStruct((NBLK, 1, NB), jnp.int32),
            jax.ShapeDtypeStruct((N, K), jnp.float32),
            jax.ShapeDtypeStruct((1, K), jnp.float32),
        ],
        compiler_params=pltpu.CompilerParams(
            dimension_semantics=("arbitrary",)),
    )(flat, x2, et, e2)

    idx = idx3.reshape(1, N)

    sc_mesh = plsc.VectorSubcoreMesh(
        core_axis_name="core", subcore_axis_name="subcore")
    gather = pl.kernel(
        _sc_gather,
        out_type=jax.ShapeDtypeStruct((N, D), jnp.float32),
        mesh=sc_mesh,
    )
    quantized = gather(embedding, idx)

    qst, loss, perp = pl.pallas_call(
        _finalize,
        in_specs=[
            pl.BlockSpec((N, D), lambda: (0, 0)),
            pl.BlockSpec((N, D), lambda: (0, 0)),
            pl.BlockSpec((1, K), lambda: (0, 0)),
        ],
        out_specs=[
            pl.BlockSpec((N, D), lambda: (0, 0)),
            pl.BlockSpec((1, 1), lambda: (0, 0)),
            pl.BlockSpec((1, 1), lambda: (0, 0)),
        ],
        out_shape=[
            jax.ShapeDtypeStruct((N, D), jnp.float32),
            jax.ShapeDtypeStruct((1, 1), jnp.float32),
            jax.ShapeDtypeStruct((1, 1), jnp.float32),
        ],
    )(flat, quantized, counts)

    loss = loss[0, 0]
    perplexity = perp[0, 0]
    qst_nchw = jnp.transpose(qst.reshape(input_shape), (0, 3, 1, 2))
    return (loss, qst_nchw, perplexity, enc)
